# baseline (speedup 1.0000x reference)
import jax
import jax.numpy as jnp
from jax import lax
from jax.experimental import pallas as pl
from jax.experimental.pallas import tpu as pltpu
from jax.experimental.pallas import tpu_sc as plsc

N_NODES = 10000
N_EDGES = 320000
D_IN = 128
D_OUT = 128

NC = 2
NS = 16
NW = NC * NS
CH = 96
NCH = 106
EPW = NCH * CH
E_PAD = EPW * NW - N_EDGES
NP = 10016
RPT = 624
REM = NP - NS * RPT
ZR = 48


def _sc_aggregate_body(feat_hbm, srcp_hbm, dstp_hbm, parts_hbm, degp_hbm,
                       didx, sidx0, sidx1, rows0, rows1, deg_v, agg,
                       isem0, isem1, gsem0, gsem1, ssem0, ssem1):
    cid = lax.axis_index("c")
    sid = lax.axis_index("s")
    wid = cid * NS + sid

    pltpu.sync_copy(dstp_hbm.at[wid, pl.ds(0, NCH)], didx.at[pl.ds(0, NCH)])
    jvec = jnp.full((16,), N_NODES, jnp.int32)
    for j in range(CH // 16):
        didx[NCH, pl.ds(j * 16, 16)] = jvec

    zvec = jnp.zeros((16,), jnp.float32)

    def _zero_row(i, _):
        for j in range(D_IN // 16):
            rows0[i, pl.ds(j * 16, 16)] = zvec
        return 0

    lax.fori_loop(0, ZR, _zero_row, 0)

    def _zero_deg(i, _):
        deg_v[pl.ds(i * 16, 16)] = zvec
        return 0

    lax.fori_loop(0, NP // 16, _zero_deg, 0)

    for k in range(RPT // ZR):
        pltpu.sync_copy(rows0.at[pl.ds(0, ZR)],
                        agg.at[pl.ds(sid * RPT + k * ZR, ZR)])

    @pl.when(sid == NS - 1)
    def _zero_rem():
        pltpu.sync_copy(rows0.at[pl.ds(0, REM)], agg.at[pl.ds(NS * RPT, REM)])

    plsc.subcore_barrier()

    ones16 = jnp.ones((16,), jnp.float32)

    pltpu.sync_copy(srcp_hbm.at[wid, 0], sidx0)
    pltpu.async_copy(srcp_hbm.at[wid, 1], sidx1, isem1)
    pltpu.async_copy(rows0, agg.at[didx.at[NCH]], ssem1, add=True)
    pltpu.async_copy(feat_hbm.at[sidx0], rows0, gsem0)

    def _pair(t, _):
        for (b, sidx, isem, rows, gsem, ssem,
             sidx_n, isem_n, rows_n, gsem_n, ssem_n) in (
                (0, sidx0, isem0, rows0, gsem0, ssem0,
                 sidx1, isem1, rows1, gsem1, ssem1),
                (1, sidx1, isem1, rows1, gsem1, ssem1,
                 sidx0, isem0, rows0, gsem0, ssem0)):
            c = 2 * t + b
            for g in range(CH // 16):
                d16 = didx[c, pl.ds(g * 16, 16)]
                plsc.addupdate_scatter(deg_v, [d16], ones16)
            pltpu.make_async_copy(srcp_hbm.at[wid, c], sidx_n, isem_n).wait()
            pltpu.make_async_copy(rows_n, agg.at[didx.at[NCH]], ssem_n).wait()
            pltpu.async_copy(feat_hbm.at[sidx_n], rows_n, gsem_n)
            pltpu.make_async_copy(feat_hbm.at[sidx_n], rows, gsem).wait()
            pltpu.async_copy(srcp_hbm.at[wid, c + 2], sidx, isem)
            pltpu.async_copy(rows, agg.at[didx.at[c]], ssem, add=True)
        return 0

    lax.fori_loop(0, NCH // 2, _pair, 0)

    pltpu.make_async_copy(feat_hbm.at[sidx0], rows0, gsem0).wait()
    pltpu.make_async_copy(rows1, agg.at[didx.at[NCH]], ssem1).wait()
    pltpu.make_async_copy(srcp_hbm.at[wid, 0], sidx1, isem1).wait()

    plsc.subcore_barrier()

    pltpu.sync_copy(agg.at[pl.ds(sid * RPT, RPT)],
                    parts_hbm.at[cid, pl.ds(sid * RPT, RPT)])

    @pl.when(sid == NS - 1)
    def _write_rem():
        pltpu.sync_copy(agg.at[pl.ds(NS * RPT, REM)],
                        parts_hbm.at[cid, pl.ds(NS * RPT, REM)])

    pltpu.sync_copy(deg_v, degp_hbm.at[wid])


def _sc_aggregate(feat, srcp, dstp):
    mesh = plsc.VectorSubcoreMesh(core_axis_name="c", subcore_axis_name="s")
    return pl.kernel(
        _sc_aggregate_body,
        out_type=(jax.ShapeDtypeStruct((NC, NP, D_IN), jnp.float32),
                  jax.ShapeDtypeStruct((NW, NP), jnp.float32)),
        mesh=mesh,
        compiler_params=pltpu.CompilerParams(needs_layout_passes=False),
        scratch_types=[
            pltpu.VMEM((NCH + 1, CH), jnp.int32),
            pltpu.VMEM((CH,), jnp.int32),
            pltpu.VMEM((CH,), jnp.int32),
            pltpu.VMEM((CH, D_IN), jnp.float32),
            pltpu.VMEM((CH, D_IN), jnp.float32),
            pltpu.VMEM((NP,), jnp.float32),
            pltpu.VMEM_SHARED((NP, D_IN), jnp.float32),
            pltpu.SemaphoreType.DMA,
            pltpu.SemaphoreType.DMA,
            pltpu.SemaphoreType.DMA,
            pltpu.SemaphoreType.DMA,
            pltpu.SemaphoreType.DMA,
            pltpu.SemaphoreType.DMA,
        ],
    )(feat, srcp, dstp)


BR = 1000


def _tc_epilogue_body(feat_ref, parts_ref, degp_ref, ws_ref, wn_ref, b_ref,
                      out_ref):
    agg = parts_ref[0] + parts_ref[1]
    deg = jnp.sum(degp_ref[...], axis=1)[:, None]
    h_neigh = agg / jnp.maximum(deg, 1.0)
    acc = jnp.dot(feat_ref[...], ws_ref[...], preferred_element_type=jnp.float32)
    acc = acc + jnp.dot(h_neigh, wn_ref[...], preferred_element_type=jnp.float32)
    out_ref[...] = jnp.maximum(acc + b_ref[...], 0.0)


def _tc_epilogue(feat, parts, deg_parts_t, W_self, W_neigh, b2d):
    return pl.pallas_call(
        _tc_epilogue_body,
        grid=(N_NODES // BR,),
        in_specs=[
            pl.BlockSpec((BR, D_IN), lambda i: (i, 0)),
            pl.BlockSpec((NC, BR, D_IN), lambda i: (0, i, 0)),
            pl.BlockSpec((BR, NW), lambda i: (i, 0)),
            pl.BlockSpec((D_IN, D_OUT), lambda i: (0, 0)),
            pl.BlockSpec((D_IN, D_OUT), lambda i: (0, 0)),
            pl.BlockSpec((1, D_OUT), lambda i: (0, 0)),
        ],
        out_specs=pl.BlockSpec((BR, D_OUT), lambda i: (i, 0)),
        out_shape=jax.ShapeDtypeStruct((N_NODES, D_OUT), jnp.float32),
    )(feat, parts, deg_parts_t, W_self, W_neigh, b2d)


@jax.jit
def kernel(feat, edge_index, W_self, W_neigh, b):
    src = edge_index[0].astype(jnp.int32)
    dst = edge_index[1].astype(jnp.int32)
    junk = N_NODES + jnp.arange(E_PAD, dtype=jnp.int32) % (NP - N_NODES)
    srcp = jnp.concatenate([src, jnp.zeros((E_PAD,), jnp.int32)])
    srcp = jnp.pad(srcp.reshape(NW, NCH, CH), ((0, 0), (0, 2), (0, 0)))
    dstp = jnp.concatenate([dst, junk]).reshape(NW, NCH, CH)
    parts, deg_parts = _sc_aggregate(feat, srcp, dstp)
    return _tc_epilogue(feat, parts, deg_parts.T, W_self, W_neigh,
                        b.reshape(1, D_OUT))
